# baseline (speedup 1.0000x reference)
import functools

import jax
import jax.numpy as jnp
from jax import lax
from jax.experimental import pallas as pl
from jax.experimental.pallas import tpu as pltpu
from jax.experimental.pallas import tpu_sc as plsc

F32 = jnp.float32
NEG_INF = float("-inf")


def _mm(a, b):
    return lax.dot_general(a, b, (((1,), (0,)), ((), ())),
                           preferred_element_type=F32)


def _leaky(x):
    return jnp.maximum(x, 0.01 * x)


def _sig(x):
    return 1.0 / (1.0 + jnp.exp(-x))


def _node_tables(nf, gu, bu, gp, gpb, gq, gqb):
    n = nf.shape[0]
    blk = 400

    def body(nf_r, gu_r, bu_r, gp_r, gpb_r, gq_r, gqb_r, unf_o, pp_o, qq_o):
        x = nf_r[...]
        unf_o[...] = _mm(x, gu_r[...]) + bu_r[...]
        pp_o[...] = _mm(x, gp_r[...]) + gpb_r[...]
        qq_o[...] = _mm(x, gq_r[...]) + gqb_r[...]

    z = lambda i: (0, 0)
    return pl.pallas_call(
        body,
        grid=(n // blk,),
        in_specs=[pl.BlockSpec((blk, 128), lambda i: (i, 0)),
                  pl.BlockSpec((128, 64), z), pl.BlockSpec((1, 64), z),
                  pl.BlockSpec((128, 128), z), pl.BlockSpec((1, 128), z),
                  pl.BlockSpec((128, 128), z), pl.BlockSpec((1, 128), z)],
        out_specs=[pl.BlockSpec((blk, 64), lambda i: (i, 0)),
                   pl.BlockSpec((blk, 128), lambda i: (i, 0)),
                   pl.BlockSpec((blk, 128), lambda i: (i, 0))],
        out_shape=[jax.ShapeDtypeStruct((n, 64), F32),
                   jax.ShapeDtypeStruct((n, 128), F32),
                   jax.ShapeDtypeStruct((n, 128), F32)],
    )(nf, gu, bu, gp, gpb, gq, gqb)


def _edge_tables(ef, ge, geb, n_real):
    e = ef.shape[0]
    blk = 4096

    def body(ef_r, ge_r, geb_r, rr_o):
        i = pl.program_id(0)
        rr = _mm(ef_r[...], ge_r[...]) + geb_r[...]
        rows = i * blk + lax.broadcasted_iota(jnp.int32, (blk, 80), 0)
        cols = lax.broadcasted_iota(jnp.int32, (blk, 80), 1)
        pad = (rows >= n_real) & (cols == 64)
        rr_o[...] = jnp.where(pad, -1e9, rr)

    z = lambda i: (0, 0)
    return pl.pallas_call(
        body,
        grid=(e // blk,),
        in_specs=[pl.BlockSpec((blk, 16), lambda i: (i, 0)),
                  pl.BlockSpec((16, 80), z), pl.BlockSpec((1, 80), z)],
        out_specs=pl.BlockSpec((blk, 80), lambda i: (i, 0)),
        out_shape=jax.ShapeDtypeStruct((e, 80), F32),
    )(ef, ge, geb)


def _node_update(h, unf, w2, b2, w3, b3, w4, b4, wr, br):
    n = unf.shape[0]
    blk = 400

    def body(h_r, unf_r, w2_r, b2_r, w3_r, b3_r, w4_r, b4_r, wr_r, br_r,
             nf_o, wn_o):
        hs = h_r[0] + h_r[1]
        agg = _mm(hs[:, :64], w2_r[...]) + hs[:, 64:65] * b2_r[...]
        x = jnp.concatenate([unf_r[...], agg], axis=1)
        t = _leaky(_mm(x, w3_r[...]) + b3_r[...])
        nfv = _mm(t, w4_r[...]) + b4_r[...]
        wn = _sig(_mm(nfv, wr_r[...]) + br_r[...])
        nf_o[...] = nfv
        wn_o[...] = wn * nfv

    z = lambda i: (0, 0)
    return pl.pallas_call(
        body,
        grid=(n // blk,),
        in_specs=[pl.BlockSpec((2, blk, 80), lambda i: (0, i, 0)),
                  pl.BlockSpec((blk, 64), lambda i: (i, 0)),
                  pl.BlockSpec((64, 64), z), pl.BlockSpec((1, 64), z),
                  pl.BlockSpec((128, 64), z), pl.BlockSpec((1, 64), z),
                  pl.BlockSpec((64, 64), z), pl.BlockSpec((1, 64), z),
                  pl.BlockSpec((64, 1), z), pl.BlockSpec((1, 1), z)],
        out_specs=[pl.BlockSpec((blk, 64), lambda i: (i, 0)),
                   pl.BlockSpec((blk, 64), lambda i: (i, 0))],
        out_shape=[jax.ShapeDtypeStruct((n, 64), F32),
                   jax.ShapeDtypeStruct((n, 64), F32)],
    )(h, unf, w2, b2, w3, b3, w4, b4, wr, br)


def _combine_max(m_ref):
    m0 = m_ref[0]
    m1 = m_ref[1]
    for k in range(2, 32, 2):
        m0 = jnp.maximum(m0, m_ref[k])
        m1 = jnp.maximum(m1, m_ref[k + 1])
    m = jnp.concatenate([m0, m1], axis=1)
    return jnp.where(m > NEG_INF, m, 0.0)


def _fg_update(fsum, fmax, ff, encw, encb, w1, b1, w2, b2, gw, gb,
               w3, b3, w4, b4, wr, br):
    f = ff.shape[0]
    blk = 400

    def body(s_r, m_r, ff_r, encw_r, encb_r, w1_r, b1_r, w2_r, b2_r,
             gw_r, gb_r, w3_r, b3_r, w4_r, b4_r, wr_r, br_r, ff_o, wf_o):
        a_sum = s_r[0] + s_r[1]
        a_max = _combine_max(m_r)
        uff = _mm(ff_r[...], encw_r[...]) + encb_r[...]
        ctx = jnp.concatenate([a_sum, a_max, uff, uff], axis=1)
        t = _leaky(_mm(ctx, w1_r[...]) + b1_r[...])
        msg = _mm(t, w2_r[...]) + b2_r[...]
        gate = _sig(_mm(ctx, gw_r[...]) + gb_r[...])
        x = jnp.concatenate([uff, gate * msg, a_sum], axis=1)
        t2 = _leaky(_mm(x, w3_r[...]) + b3_r[...])
        ffv = _mm(t2, w4_r[...]) + b4_r[...]
        wf = _sig(_mm(ffv, wr_r[...]) + br_r[...])
        ff_o[...] = ffv
        wf_o[...] = wf * ffv

    z = lambda i: (0, 0)
    return pl.pallas_call(
        body,
        grid=(f // blk,),
        in_specs=[pl.BlockSpec((2, blk, 64), lambda i: (0, i, 0)),
                  pl.BlockSpec((32, blk, 32), lambda i: (0, i, 0)),
                  pl.BlockSpec((blk, 128), lambda i: (i, 0)),
                  pl.BlockSpec((128, 64), z), pl.BlockSpec((1, 64), z),
                  pl.BlockSpec((256, 64), z), pl.BlockSpec((1, 64), z),
                  pl.BlockSpec((64, 64), z), pl.BlockSpec((1, 64), z),
                  pl.BlockSpec((256, 64), z), pl.BlockSpec((1, 64), z),
                  pl.BlockSpec((192, 64), z), pl.BlockSpec((1, 64), z),
                  pl.BlockSpec((64, 64), z), pl.BlockSpec((1, 64), z),
                  pl.BlockSpec((64, 1), z), pl.BlockSpec((1, 1), z)],
        out_specs=[pl.BlockSpec((blk, 64), lambda i: (i, 0)),
                   pl.BlockSpec((blk, 64), lambda i: (i, 0))],
        out_shape=[jax.ShapeDtypeStruct((f, 64), F32),
                   jax.ShapeDtypeStruct((f, 64), F32)],
    )(fsum, fmax, ff, encw, encb, w1, b1, w2, b2, gw, gb,
      w3, b3, w4, b4, wr, br)


def _readout(nsum, nmax, gsum, gmax, w1, b1, w2, b2):
    g = 256

    def body(ns_r, nm_r, fs_r, fm_r, w1_r, b1_r, w2_r, b2_r, y_o):
        n_wsum = ns_r[0] + ns_r[1]
        n_max = _combine_max(nm_r)
        f_wsum = fs_r[0] + fs_r[1]
        f_max = _combine_max(fm_r)
        ro = jnp.concatenate([n_wsum, n_max, f_wsum, f_max], axis=1)
        t = _leaky(_mm(ro, w1_r[...]) + b1_r[...])
        y_o[...] = _mm(t, w2_r[...]) + b2_r[...]

    z = lambda i: (0, 0)
    return pl.pallas_call(
        body,
        grid=(1,),
        in_specs=[pl.BlockSpec((2, g, 64), lambda i: (0, 0, 0)),
                  pl.BlockSpec((32, g, 32), lambda i: (0, 0, 0)),
                  pl.BlockSpec((2, g, 64), lambda i: (0, 0, 0)),
                  pl.BlockSpec((32, g, 32), lambda i: (0, 0, 0)),
                  pl.BlockSpec((256, 64), z), pl.BlockSpec((1, 64), z),
                  pl.BlockSpec((64, 1), z), pl.BlockSpec((1, 1), z)],
        out_specs=pl.BlockSpec((g, 1), lambda i: (0, 0)),
        out_shape=jax.ShapeDtypeStruct((g, 1), F32),
    )(nsum, nmax, gsum, gmax, w1, b1, w2, b2)


_MESH = dict(core_axis_name="c", subcore_axis_name="s")


@functools.cache
def _edge_sc(n_nodes, n_edges):
    eb = 128
    ept = n_edges // 32
    nchunks = ept // eb
    nacc = ((n_nodes + 2047) // 2048) * 2048
    zrows = nacc // 16
    zb = 64
    zc = zrows // zb

    @functools.partial(
        pl.kernel,
        out_type=jax.ShapeDtypeStruct((2, nacc, 80), F32),
        mesh=plsc.VectorSubcoreMesh(**_MESH),
        scratch_types=[
            pltpu.VMEM_SHARED((nacc, 80), F32),
            pltpu.VMEM((eb,), jnp.int32),
            pltpu.VMEM((eb,), jnp.int32),
            pltpu.VMEM((eb, 80), F32),
            pltpu.VMEM((eb, 128), F32),
            pltpu.VMEM((eb, 128), F32),
            pltpu.VMEM((eb, 80), F32),
            pltpu.VMEM((64, 80), F32),
            pltpu.SemaphoreType.DMA,
        ],
    )
    def k(pp_hbm, qq_hbm, rr_hbm, si_hbm, di_hbm, out_hbm,
          acc, sidx, didx, rbuf, pbuf, qbuf, vbuf, zbuf, sem):
        c = lax.axis_index("c")
        s = lax.axis_index("s")
        wid = s * 2 + c
        zv = jnp.zeros((16,), F32)

        def zfill(i, carry):
            zbuf[i // 5, pl.ds((i % 5) * 16, 16)] = zv
            return carry
        lax.fori_loop(0, zb * 5, zfill, None)

        def zcopy(j, carry):
            pltpu.sync_copy(zbuf, acc.at[pl.ds(s * zrows + j * zb, zb), :])
            return carry
        lax.fori_loop(0, zc, zcopy, None)
        plsc.subcore_barrier()

        base_t = wid * ept
        zi16 = jnp.zeros((16,), jnp.int32)

        def chunk(ci, carry):
            e0 = base_t + ci * eb
            pltpu.sync_copy(si_hbm.at[pl.ds(e0, eb)], sidx)
            pltpu.sync_copy(di_hbm.at[pl.ds(e0, eb)], didx)
            pltpu.sync_copy(rr_hbm.at[pl.ds(e0, eb), :], rbuf)
            cp1 = pltpu.async_copy(pp_hbm.at[sidx], pbuf, sem)
            cp2 = pltpu.async_copy(qq_hbm.at[didx], qbuf, sem)
            cp1.wait()
            cp2.wait()

            def edge(i, carry2):
                t = (pbuf[i, pl.ds(64, 16)] + qbuf[i, pl.ds(64, 16)]
                     + rbuf[i, pl.ds(64, 16)])
                a = 1.0 / (1.0 + jnp.exp(-t))
                vbuf[i, pl.ds(64, 16)] = a
                attn = a[zi16]
                for g in range(4):
                    hv = (pbuf[i, pl.ds(g * 16, 16)]
                          + qbuf[i, pl.ds(g * 16, 16)]
                          + rbuf[i, pl.ds(g * 16, 16)])
                    hv = jnp.maximum(hv, 0.01 * hv)
                    vbuf[i, pl.ds(g * 16, 16)] = attn * hv
                return carry2
            lax.fori_loop(0, eb, edge, None)
            pltpu.sync_copy(vbuf, acc.at[didx], add=True)
            return carry
        lax.fori_loop(0, nchunks, chunk, None)
        plsc.subcore_barrier()

        def dump(j, carry):
            r0 = s * zrows + j * zb
            pltpu.sync_copy(acc.at[pl.ds(r0, zb), :],
                            out_hbm.at[c, pl.ds(r0, zb), :])
            return carry
        lax.fori_loop(0, zc, dump, None)

    return k


@functools.cache
def _segred(np_, s_, m_):
    sb = 64
    n_schunk = np_ // 32 // sb
    mb = 128
    n_mchunk = np_ // 16 // mb
    sacc = ((s_ + 255) // 256) * 256
    zrows = sacc // 16
    m4 = m_ // 4

    @functools.partial(
        pl.kernel,
        out_type=[jax.ShapeDtypeStruct((2, sacc, 64), F32),
                  jax.ShapeDtypeStruct((32, m4, 128), F32)],
        mesh=plsc.VectorSubcoreMesh(**_MESH),
        scratch_types=[
            pltpu.VMEM_SHARED((sacc, 64), F32),
            pltpu.VMEM((m4, 128), F32),
            pltpu.VMEM((sb, 64), F32),
            pltpu.VMEM((mb, 32), F32),
            pltpu.VMEM((sb,), jnp.int32),
            pltpu.VMEM((mb,), jnp.int32),
            pltpu.VMEM((128, 64), F32),
            pltpu.SemaphoreType.DMA,
        ],
    )
    def k(xs_hbm, ids_s_hbm, xml_hbm, xmr_hbm, ids_m_hbm,
          out_sum, out_max, sum_acc, max_acc, xs_v, xm_v, ids_s_v, ids_m_v,
          zbuf, sem):
        c = lax.axis_index("c")
        s = lax.axis_index("s")
        wid = s * 2 + c
        zv = jnp.zeros((16,), F32)

        def zfill(i, carry):
            zbuf[i // 4, pl.ds((i % 4) * 16, 16)] = zv
            return carry
        lax.fori_loop(0, zrows * 4, zfill, None)
        pltpu.sync_copy(zbuf.at[pl.ds(0, zrows), :],
                        sum_acc.at[pl.ds(s * zrows, zrows), :])

        ninf = jnp.full((16,), NEG_INF, F32)

        def mfill(i, carry):
            max_acc[i // 8, pl.ds((i % 8) * 16, 16)] = ninf
            return carry
        lax.fori_loop(0, m4 * 8, mfill, None)
        plsc.subcore_barrier()

        sum_base = wid * (np_ // 32)

        def schunk(ci, carry):
            r0 = sum_base + ci * sb
            pltpu.sync_copy(ids_s_hbm.at[pl.ds(r0, sb)], ids_s_v)
            pltpu.sync_copy(xs_hbm.at[pl.ds(r0, sb), :], xs_v)
            pltpu.sync_copy(xs_v, sum_acc.at[ids_s_v], add=True)
            return carry
        lax.fori_loop(0, n_schunk, schunk, None)

        max_base = s * (np_ // 16)

        def mchunk(ci, carry):
            r0 = max_base + ci * mb
            pltpu.sync_copy(ids_m_hbm.at[pl.ds(r0, mb)], ids_m_v)

            @pl.when(c == 0)
            def _l():
                pltpu.sync_copy(xml_hbm.at[pl.ds(r0, mb), :], xm_v)

            @pl.when(c == 1)
            def _r():
                pltpu.sync_copy(xmr_hbm.at[pl.ds(r0, mb), :], xm_v)

            def rgrp(gi, carry2):
                idv = ids_m_v[pl.ds(gi * 16, 16)]
                for i in range(16):
                    idx = idv[i]
                    row = idx // 4
                    lb = (idx % 4) * 32
                    r = gi * 16 + i
                    for g in range(2):
                        old = max_acc[row, pl.ds(lb + g * 16, 16)]
                        xv = xm_v[r, pl.ds(g * 16, 16)]
                        max_acc[row, pl.ds(lb + g * 16, 16)] = (
                            jnp.maximum(old, xv))
                return carry2
            lax.fori_loop(0, mb // 16, rgrp, None)
            return carry
        lax.fori_loop(0, n_mchunk, mchunk, None)
        plsc.subcore_barrier()

        pltpu.sync_copy(sum_acc.at[pl.ds(s * zrows, zrows), :],
                        out_sum.at[c, pl.ds(s * zrows, zrows), :])
        pltpu.sync_copy(max_acc, out_max.at[wid])

    return k


def kernel(nf, ef, ff, edge_index, fg_assign, node_graph_ids,
           fg_graph_ids, params):
    p = params
    nh = 64
    n = nf.shape[0]
    e = ef.shape[0]
    f = ff.shape[0]

    u = p['am_W1'] @ p['am_W2']
    zpad63 = jnp.zeros((nh, 63), F32)
    zpad15 = jnp.zeros((nh, 15), F32)
    wp = jnp.concatenate([p['em_W1'][:nh], u[:nh], zpad63], axis=1)
    wq = jnp.concatenate([p['em_W1'][nh:2 * nh], u[nh:2 * nh], zpad63], axis=1)
    we = jnp.concatenate([p['em_W1'][2 * nh:], u[2 * nh:], zpad15], axis=1)
    gu, bu = p['enc_node_W'], p['enc_node_b']
    gp, gpb = gu @ wp, bu @ wp
    gq, gqb = gu @ wq, bu @ wq
    ge = p['enc_edge_W'] @ we
    ce = p['am_b1'] @ p['am_W2'] + p['am_b2']
    geb = p['enc_edge_b'] @ we + jnp.concatenate(
        [p['em_b1'], ce, jnp.zeros((15,), F32)])
    gate_w = p['fam_W1'] @ p['fam_W2']
    gate_b = p['fam_b1'] @ p['fam_W2'] + p['fam_b2']

    r2 = lambda v: v.reshape(1, -1)

    unf, pp, qq = _node_tables(nf, gu, r2(bu), gp, r2(gpb), gq, r2(gqb))
    epad = ((e + 4095) // 4096) * 4096
    ef_p = jnp.pad(ef, ((0, epad - e), (0, 0)))
    rr = _edge_tables(ef_p, ge, r2(geb), e)
    si = jnp.pad(edge_index[0].astype(jnp.int32), (0, epad - e))
    di = jnp.pad(edge_index[1].astype(jnp.int32), (0, epad - e))
    h = _edge_sc(n, epad)(pp, qq, rr, si, di)

    new_nf, wn = _node_update(
        h, unf, p['em_W2'], r2(p['em_b2']), p['nm_W1'], r2(p['nm_b1']),
        p['nm_W2'], r2(p['nm_b2']), p['rd_node_W'], r2(p['rd_node_b']))

    npad = 10240 - n
    nf0 = jnp.pad(new_nf, ((0, npad), (0, 0)))
    wn0 = jnp.pad(wn, ((0, npad), (0, 0)))
    nfl = jnp.pad(new_nf[:, :32], ((0, npad), (0, 0)),
                  constant_values=NEG_INF)
    nfr = jnp.pad(new_nf[:, 32:], ((0, npad), (0, 0)),
                  constant_values=NEG_INF)
    fgi = jnp.pad(fg_assign.astype(jnp.int32), (0, npad))
    ngi = jnp.pad(node_graph_ids.astype(jnp.int32), (0, npad))
    fsum, fmax = _segred(10240, f, f)(nf0, fgi, nfl, nfr, fgi)
    fsum = fsum[:, :f]
    fmax = fmax.reshape(32, f, 32)
    nsum, nmax = _segred(10240, 256, 256)(wn0, ngi, nfl, nfr, ngi)
    nmax = nmax.reshape(32, 256, 32)

    new_ff, wf = _fg_update(
        fsum, fmax, ff, p['enc_fg_W'], r2(p['enc_fg_b']),
        p['fem_W1'], r2(p['fem_b1']), p['fem_W2'], r2(p['fem_b2']),
        gate_w, r2(gate_b), p['fnm_W1'], r2(p['fnm_b1']),
        p['fnm_W2'], r2(p['fnm_b2']), p['rd_fg_W'], r2(p['rd_fg_b']))

    fpad = 2048 - f
    wf0 = jnp.pad(wf, ((0, fpad), (0, 0)))
    ffl = jnp.pad(new_ff[:, :32], ((0, fpad), (0, 0)),
                  constant_values=NEG_INF)
    ffr = jnp.pad(new_ff[:, 32:], ((0, fpad), (0, 0)),
                  constant_values=NEG_INF)
    fgg = jnp.pad(fg_graph_ids.astype(jnp.int32), (0, fpad))
    gsum, gmax = _segred(2048, 256, 256)(wf0, fgg, ffl, ffr, fgg)
    gmax = gmax.reshape(32, 256, 32)

    return _readout(nsum, nmax, gsum, gmax,
                    p['reg_W1'], r2(p['reg_b1']),
                    p['reg_W2'], r2(p['reg_b2']))


# speedup vs baseline: 1.3443x; 1.3443x over previous
import functools

import jax
import jax.numpy as jnp
from jax import lax
from jax.experimental import pallas as pl
from jax.experimental.pallas import tpu as pltpu
from jax.experimental.pallas import tpu_sc as plsc

F32 = jnp.float32
NEG_INF = float("-inf")


def _mm(a, b):
    return lax.dot_general(a, b, (((1,), (0,)), ((), ())),
                           preferred_element_type=F32)


def _leaky(x):
    return jnp.maximum(x, 0.01 * x)


def _sig(x):
    return 1.0 / (1.0 + jnp.exp(-x))


def _node_tables(nf, gu, bu, gp, gpb, gq, gqb):
    n = nf.shape[0]
    blk = 400

    def body(nf_r, gu_r, bu_r, gp_r, gpb_r, gq_r, gqb_r, unf_o, pp_o, qq_o):
        x = nf_r[...]
        unf_o[...] = _mm(x, gu_r[...]) + bu_r[...]
        pp_o[...] = _mm(x, gp_r[...]) + gpb_r[...]
        qq_o[...] = _mm(x, gq_r[...]) + gqb_r[...]

    z = lambda i: (0, 0)
    return pl.pallas_call(
        body,
        grid=(n // blk,),
        in_specs=[pl.BlockSpec((blk, 128), lambda i: (i, 0)),
                  pl.BlockSpec((128, 64), z), pl.BlockSpec((1, 64), z),
                  pl.BlockSpec((128, 128), z), pl.BlockSpec((1, 128), z),
                  pl.BlockSpec((128, 128), z), pl.BlockSpec((1, 128), z)],
        out_specs=[pl.BlockSpec((blk, 64), lambda i: (i, 0)),
                   pl.BlockSpec((blk, 128), lambda i: (i, 0)),
                   pl.BlockSpec((blk, 128), lambda i: (i, 0))],
        out_shape=[jax.ShapeDtypeStruct((n, 64), F32),
                   jax.ShapeDtypeStruct((n, 128), F32),
                   jax.ShapeDtypeStruct((n, 128), F32)],
    )(nf, gu, bu, gp, gpb, gq, gqb)


def _edge_tables(ef, ge, geb):
    e = ef.shape[0]
    blk = 3200

    def body(ef_r, ge_r, geb_r, rr_o):
        rr_o[...] = _mm(ef_r[...], ge_r[...]) + geb_r[...]

    z = lambda i: (0, 0)
    return pl.pallas_call(
        body,
        grid=(e // blk,),
        in_specs=[pl.BlockSpec((blk, 16), lambda i: (i, 0)),
                  pl.BlockSpec((16, 128), z), pl.BlockSpec((1, 128), z)],
        out_specs=pl.BlockSpec((blk, 128), lambda i: (i, 0)),
        out_shape=jax.ShapeDtypeStruct((e, 128), F32),
    )(ef, ge, geb)


def _node_update(h, unf, w2, b2, w3, b3, w4, b4, wr, br):
    n = unf.shape[0]
    blk = 400

    def body(h_r, unf_r, w2_r, b2_r, w3_r, b3_r, w4_r, b4_r, wr_r, br_r,
             nf_o, wn_o):
        hs = h_r[0] + h_r[1]
        agg = _mm(hs[:, :64], w2_r[...]) + hs[:, 64:65] * b2_r[...]
        x = jnp.concatenate([unf_r[...], agg], axis=1)
        t = _leaky(_mm(x, w3_r[...]) + b3_r[...])
        nfv = _mm(t, w4_r[...]) + b4_r[...]
        wn = _sig(_mm(nfv, wr_r[...]) + br_r[...])
        nf_o[...] = nfv
        wn_o[...] = wn * nfv

    z = lambda i: (0, 0)
    return pl.pallas_call(
        body,
        grid=(n // blk,),
        in_specs=[pl.BlockSpec((2, blk, 80), lambda i: (0, i, 0)),
                  pl.BlockSpec((blk, 64), lambda i: (i, 0)),
                  pl.BlockSpec((64, 64), z), pl.BlockSpec((1, 64), z),
                  pl.BlockSpec((128, 64), z), pl.BlockSpec((1, 64), z),
                  pl.BlockSpec((64, 64), z), pl.BlockSpec((1, 64), z),
                  pl.BlockSpec((64, 1), z), pl.BlockSpec((1, 1), z)],
        out_specs=[pl.BlockSpec((blk, 64), lambda i: (i, 0)),
                   pl.BlockSpec((blk, 64), lambda i: (i, 0))],
        out_shape=[jax.ShapeDtypeStruct((n, 64), F32),
                   jax.ShapeDtypeStruct((n, 64), F32)],
    )(h, unf, w2, b2, w3, b3, w4, b4, wr, br)


def _combine_max(m_ref):
    m0 = m_ref[0]
    m1 = m_ref[1]
    for k in range(2, 32, 2):
        m0 = jnp.maximum(m0, m_ref[k])
        m1 = jnp.maximum(m1, m_ref[k + 1])
    m = jnp.concatenate([m0, m1], axis=1)
    return jnp.where(m > NEG_INF, m, 0.0)


def _fg_update(fsum, fmax, ff, encw, encb, w1, b1, w2, b2, gw, gb,
               w3, b3, w4, b4, wr, br):
    f = ff.shape[0]
    blk = 400

    def body(s_r, m_r, ff_r, encw_r, encb_r, w1_r, b1_r, w2_r, b2_r,
             gw_r, gb_r, w3_r, b3_r, w4_r, b4_r, wr_r, br_r, ff_o, wf_o):
        a_sum = s_r[0] + s_r[1]
        a_max = _combine_max(m_r)
        uff = _mm(ff_r[...], encw_r[...]) + encb_r[...]
        ctx = jnp.concatenate([a_sum, a_max, uff, uff], axis=1)
        t = _leaky(_mm(ctx, w1_r[...]) + b1_r[...])
        msg = _mm(t, w2_r[...]) + b2_r[...]
        gate = _sig(_mm(ctx, gw_r[...]) + gb_r[...])
        x = jnp.concatenate([uff, gate * msg, a_sum], axis=1)
        t2 = _leaky(_mm(x, w3_r[...]) + b3_r[...])
        ffv = _mm(t2, w4_r[...]) + b4_r[...]
        wf = _sig(_mm(ffv, wr_r[...]) + br_r[...])
        ff_o[...] = ffv
        wf_o[...] = wf * ffv

    z = lambda i: (0, 0)
    return pl.pallas_call(
        body,
        grid=(f // blk,),
        in_specs=[pl.BlockSpec((2, blk, 64), lambda i: (0, i, 0)),
                  pl.BlockSpec((32, blk, 32), lambda i: (0, i, 0)),
                  pl.BlockSpec((blk, 128), lambda i: (i, 0)),
                  pl.BlockSpec((128, 64), z), pl.BlockSpec((1, 64), z),
                  pl.BlockSpec((256, 64), z), pl.BlockSpec((1, 64), z),
                  pl.BlockSpec((64, 64), z), pl.BlockSpec((1, 64), z),
                  pl.BlockSpec((256, 64), z), pl.BlockSpec((1, 64), z),
                  pl.BlockSpec((192, 64), z), pl.BlockSpec((1, 64), z),
                  pl.BlockSpec((64, 64), z), pl.BlockSpec((1, 64), z),
                  pl.BlockSpec((64, 1), z), pl.BlockSpec((1, 1), z)],
        out_specs=[pl.BlockSpec((blk, 64), lambda i: (i, 0)),
                   pl.BlockSpec((blk, 64), lambda i: (i, 0))],
        out_shape=[jax.ShapeDtypeStruct((f, 64), F32),
                   jax.ShapeDtypeStruct((f, 64), F32)],
    )(fsum, fmax, ff, encw, encb, w1, b1, w2, b2, gw, gb,
      w3, b3, w4, b4, wr, br)


def _readout(nsum, nmax, gsum, gmax, w1, b1, w2, b2):
    g = 256

    def body(ns_r, nm_r, fs_r, fm_r, w1_r, b1_r, w2_r, b2_r, y_o):
        n_wsum = ns_r[0] + ns_r[1]
        n_max = _combine_max(nm_r)
        f_wsum = fs_r[0] + fs_r[1]
        f_max = _combine_max(fm_r)
        ro = jnp.concatenate([n_wsum, n_max, f_wsum, f_max], axis=1)
        t = _leaky(_mm(ro, w1_r[...]) + b1_r[...])
        y_o[...] = _mm(t, w2_r[...]) + b2_r[...]

    z = lambda i: (0, 0)
    return pl.pallas_call(
        body,
        grid=(1,),
        in_specs=[pl.BlockSpec((2, g, 64), lambda i: (0, 0, 0)),
                  pl.BlockSpec((32, g, 32), lambda i: (0, 0, 0)),
                  pl.BlockSpec((2, g, 64), lambda i: (0, 0, 0)),
                  pl.BlockSpec((32, g, 32), lambda i: (0, 0, 0)),
                  pl.BlockSpec((256, 64), z), pl.BlockSpec((1, 64), z),
                  pl.BlockSpec((64, 1), z), pl.BlockSpec((1, 1), z)],
        out_specs=pl.BlockSpec((g, 1), lambda i: (0, 0)),
        out_shape=jax.ShapeDtypeStruct((g, 1), F32),
    )(nsum, nmax, gsum, gmax, w1, b1, w2, b2)


_MESH = dict(core_axis_name="c", subcore_axis_name="s")


@functools.cache
def _edge_sc(n_nodes, n_edges):
    eb = 80
    ept = n_edges // 32
    nchunks = ept // eb
    nacc = ((n_nodes + 2047) // 2048) * 2048
    zrows = nacc // 16
    zb = 128
    zc = zrows // zb

    @functools.partial(
        pl.kernel,
        out_type=jax.ShapeDtypeStruct((2, nacc, 80), F32),
        mesh=plsc.VectorSubcoreMesh(**_MESH),
        scratch_types=[
            pltpu.VMEM_SHARED((nacc, 80), F32),
            pltpu.VMEM((eb,), jnp.int32),
            pltpu.VMEM((eb,), jnp.int32),
            pltpu.VMEM((eb,), jnp.int32),
            pltpu.VMEM((eb,), jnp.int32),
            pltpu.VMEM((eb, 128), F32),
            pltpu.VMEM((eb, 128), F32),
            pltpu.VMEM((eb, 128), F32),
            pltpu.VMEM((eb, 128), F32),
            pltpu.VMEM((eb, 128), F32),
            pltpu.VMEM((eb, 80), F32),
            pltpu.VMEM((128, 80), F32),
            pltpu.SemaphoreType.DMA,
            pltpu.SemaphoreType.DMA,
            pltpu.SemaphoreType.DMA,
            pltpu.SemaphoreType.DMA,
        ],
    )
    def k(pp_hbm, qq_hbm, rr_hbm, si_hbm, di_hbm, out_hbm,
          acc, sidx0, sidx1, didx0, didx1, rbuf, pbuf0, pbuf1,
          qbuf0, qbuf1, vbuf, zbuf, isem0, isem1, gsem0, gsem1):
        c = lax.axis_index("c")
        s = lax.axis_index("s")
        wid = s * 2 + c
        zv = jnp.zeros((16,), F32)
        sets = ((sidx0, didx0, pbuf0, qbuf0, isem0, gsem0),
                (sidx1, didx1, pbuf1, qbuf1, isem1, gsem1))

        def zfill(i, carry):
            zbuf[i // 5, pl.ds((i % 5) * 16, 16)] = zv
            return carry
        lax.fori_loop(0, zb * 5, zfill, None)

        def zcopy(j, carry):
            pltpu.sync_copy(zbuf, acc.at[pl.ds(s * zrows + j * zb, zb), :])
            return carry
        lax.fori_loop(0, zc, zcopy, None)
        plsc.subcore_barrier()

        base_t = wid * ept
        zi16 = jnp.zeros((16,), jnp.int32)

        def issue_inputs(cs, b):
            sx, dx, ism = sets[b][0], sets[b][1], sets[b][4]
            e0 = base_t + cs * eb
            pltpu.async_copy(si_hbm.at[pl.ds(e0, eb)], sx, ism)
            pltpu.async_copy(di_hbm.at[pl.ds(e0, eb)], dx, ism)

        def wait_inputs(b):
            sx, dx, ism = sets[b][0], sets[b][1], sets[b][4]
            pltpu.make_async_copy(si_hbm.at[pl.ds(0, eb)], sx, ism).wait()
            pltpu.make_async_copy(di_hbm.at[pl.ds(0, eb)], dx, ism).wait()

        def issue_gathers(b):
            sx, dx, pb, qb = (sets[b][0], sets[b][1], sets[b][2], sets[b][3])
            gsm = sets[b][5]
            pltpu.async_copy(pp_hbm.at[sx], pb, gsm)
            pltpu.async_copy(qq_hbm.at[dx], qb, gsm)

        def wait_gathers(b):
            sx, dx, pb, qb = (sets[b][0], sets[b][1], sets[b][2], sets[b][3])
            gsm = sets[b][5]
            pltpu.make_async_copy(pp_hbm.at[sx], pb, gsm).wait()
            pltpu.make_async_copy(qq_hbm.at[dx], qb, gsm).wait()

        def compute_and_scatter(ci, b):
            dx, pb, qb = sets[b][1], sets[b][2], sets[b][3]
            e0 = base_t + ci * eb
            pltpu.sync_copy(rr_hbm.at[pl.ds(e0, eb), :], rbuf)

            def edge(i, carry2):
                t = (pb[i, pl.ds(64, 16)] + qb[i, pl.ds(64, 16)]
                     + rbuf[i, pl.ds(64, 16)])
                a = 1.0 / (1.0 + jnp.exp(-t))
                vbuf[i, pl.ds(64, 16)] = a
                attn = a[zi16]
                for g in range(4):
                    hv = (pb[i, pl.ds(g * 16, 16)]
                          + qb[i, pl.ds(g * 16, 16)]
                          + rbuf[i, pl.ds(g * 16, 16)])
                    hv = jnp.maximum(hv, 0.01 * hv)
                    vbuf[i, pl.ds(g * 16, 16)] = attn * hv
                return carry2
            lax.fori_loop(0, eb, edge, None)
            pltpu.sync_copy(vbuf, acc.at[dx], add=True)

        def step(ci, b, tail=False):
            wait_gathers(b)
            if not tail:
                @pl.when(ci + 1 < nchunks)
                def _nxt():
                    wait_inputs(1 - b)
                    issue_gathers(1 - b)
            compute_and_scatter(ci, b)
            if not tail:
                @pl.when(ci + 2 < nchunks)
                def _pre():
                    issue_inputs(ci + 2, b)

        issue_inputs(0, 0)
        issue_inputs(1, 1)
        wait_inputs(0)
        issue_gathers(0)

        def pair(g, carry):
            step(2 * g, 0)
            step(2 * g + 1, 1)
            return carry
        lax.fori_loop(0, nchunks // 2, pair, None)
        if nchunks % 2:
            step(nchunks - 1, 0, tail=True)
        plsc.subcore_barrier()

        def dump(j, carry):
            r0 = s * zrows + j * zb
            pltpu.sync_copy(acc.at[pl.ds(r0, zb), :],
                            out_hbm.at[c, pl.ds(r0, zb), :])
            return carry
        lax.fori_loop(0, zc, dump, None)

    return k


@functools.cache
def _segred(np_, s_, m_):
    sb = 64
    n_schunk = np_ // 32 // sb
    mb = 128
    n_mchunk = np_ // 16 // mb
    sacc = ((s_ + 255) // 256) * 256
    zrows = sacc // 16
    m4 = m_ // 4

    @functools.partial(
        pl.kernel,
        out_type=[jax.ShapeDtypeStruct((2, sacc, 64), F32),
                  jax.ShapeDtypeStruct((32, m4, 128), F32)],
        mesh=plsc.VectorSubcoreMesh(**_MESH),
        scratch_types=[
            pltpu.VMEM_SHARED((sacc, 64), F32),
            pltpu.VMEM((m4, 128), F32),
            pltpu.VMEM((sb, 64), F32),
            pltpu.VMEM((mb, 32), F32),
            pltpu.VMEM((sb,), jnp.int32),
            pltpu.VMEM((mb,), jnp.int32),
            pltpu.VMEM((128, 64), F32),
            pltpu.SemaphoreType.DMA,
        ],
    )
    def k(xs_hbm, ids_s_hbm, xml_hbm, xmr_hbm, ids_m_hbm,
          out_sum, out_max, sum_acc, max_acc, xs_v, xm_v, ids_s_v, ids_m_v,
          zbuf, sem):
        c = lax.axis_index("c")
        s = lax.axis_index("s")
        wid = s * 2 + c
        zv = jnp.zeros((16,), F32)

        def zfill(i, carry):
            zbuf[i // 4, pl.ds((i % 4) * 16, 16)] = zv
            return carry
        lax.fori_loop(0, zrows * 4, zfill, None)
        pltpu.sync_copy(zbuf.at[pl.ds(0, zrows), :],
                        sum_acc.at[pl.ds(s * zrows, zrows), :])

        ninf = jnp.full((16,), NEG_INF, F32)

        def mfill(i, carry):
            max_acc[i // 8, pl.ds((i % 8) * 16, 16)] = ninf
            return carry
        lax.fori_loop(0, m4 * 8, mfill, None)
        plsc.subcore_barrier()

        sum_base = wid * (np_ // 32)

        def schunk(ci, carry):
            r0 = sum_base + ci * sb
            pltpu.sync_copy(ids_s_hbm.at[pl.ds(r0, sb)], ids_s_v)
            pltpu.sync_copy(xs_hbm.at[pl.ds(r0, sb), :], xs_v)
            pltpu.sync_copy(xs_v, sum_acc.at[ids_s_v], add=True)
            return carry
        lax.fori_loop(0, n_schunk, schunk, None)

        max_base = s * (np_ // 16)

        def mchunk(ci, carry):
            r0 = max_base + ci * mb
            pltpu.sync_copy(ids_m_hbm.at[pl.ds(r0, mb)], ids_m_v)

            @pl.when(c == 0)
            def _l():
                pltpu.sync_copy(xml_hbm.at[pl.ds(r0, mb), :], xm_v)

            @pl.when(c == 1)
            def _r():
                pltpu.sync_copy(xmr_hbm.at[pl.ds(r0, mb), :], xm_v)

            def rgrp(gi, carry2):
                idv = ids_m_v[pl.ds(gi * 16, 16)]
                for i in range(16):
                    idx = idv[i]
                    row = idx // 4
                    lb = (idx % 4) * 32
                    r = gi * 16 + i
                    for g in range(2):
                        old = max_acc[row, pl.ds(lb + g * 16, 16)]
                        xv = xm_v[r, pl.ds(g * 16, 16)]
                        max_acc[row, pl.ds(lb + g * 16, 16)] = (
                            jnp.maximum(old, xv))
                return carry2
            lax.fori_loop(0, mb // 16, rgrp, None)
            return carry
        lax.fori_loop(0, n_mchunk, mchunk, None)
        plsc.subcore_barrier()

        pltpu.sync_copy(sum_acc.at[pl.ds(s * zrows, zrows), :],
                        out_sum.at[c, pl.ds(s * zrows, zrows), :])
        pltpu.sync_copy(max_acc, out_max.at[wid])

    return k


def kernel(nf, ef, ff, edge_index, fg_assign, node_graph_ids,
           fg_graph_ids, params):
    p = params
    nh = 64
    n = nf.shape[0]
    e = ef.shape[0]
    f = ff.shape[0]

    u = p['am_W1'] @ p['am_W2']
    zpad63 = jnp.zeros((nh, 63), F32)
    wp = jnp.concatenate([p['em_W1'][:nh], u[:nh], zpad63], axis=1)
    wq = jnp.concatenate([p['em_W1'][nh:2 * nh], u[nh:2 * nh], zpad63], axis=1)
    we = jnp.concatenate([p['em_W1'][2 * nh:], u[2 * nh:], zpad63], axis=1)
    gu, bu = p['enc_node_W'], p['enc_node_b']
    gp, gpb = gu @ wp, bu @ wp
    gq, gqb = gu @ wq, bu @ wq
    ge = p['enc_edge_W'] @ we
    ce = p['am_b1'] @ p['am_W2'] + p['am_b2']
    geb = p['enc_edge_b'] @ we + jnp.concatenate(
        [p['em_b1'], ce, jnp.zeros((63,), F32)])
    gate_w = p['fam_W1'] @ p['fam_W2']
    gate_b = p['fam_b1'] @ p['fam_W2'] + p['fam_b2']

    r2 = lambda v: v.reshape(1, -1)

    unf, pp, qq = _node_tables(nf, gu, r2(bu), gp, r2(gpb), gq, r2(gqb))
    rr = _edge_tables(ef, ge, r2(geb))
    si = edge_index[0].astype(jnp.int32)
    di = edge_index[1].astype(jnp.int32)
    h = _edge_sc(n, e)(pp, qq, rr, si, di)

    new_nf, wn = _node_update(
        h, unf, p['em_W2'], r2(p['em_b2']), p['nm_W1'], r2(p['nm_b1']),
        p['nm_W2'], r2(p['nm_b2']), p['rd_node_W'], r2(p['rd_node_b']))

    npad = 10240 - n
    nf0 = jnp.pad(new_nf, ((0, npad), (0, 0)))
    wn0 = jnp.pad(wn, ((0, npad), (0, 0)))
    nfl = jnp.pad(new_nf[:, :32], ((0, npad), (0, 0)),
                  constant_values=NEG_INF)
    nfr = jnp.pad(new_nf[:, 32:], ((0, npad), (0, 0)),
                  constant_values=NEG_INF)
    fgi = jnp.pad(fg_assign.astype(jnp.int32), (0, npad))
    ngi = jnp.pad(node_graph_ids.astype(jnp.int32), (0, npad))
    fsum, fmax = _segred(10240, f, f)(nf0, fgi, nfl, nfr, fgi)
    fsum = fsum[:, :f]
    fmax = fmax.reshape(32, f, 32)
    nsum, nmax = _segred(10240, 256, 256)(wn0, ngi, nfl, nfr, ngi)
    nmax = nmax.reshape(32, 256, 32)

    new_ff, wf = _fg_update(
        fsum, fmax, ff, p['enc_fg_W'], r2(p['enc_fg_b']),
        p['fem_W1'], r2(p['fem_b1']), p['fem_W2'], r2(p['fem_b2']),
        gate_w, r2(gate_b), p['fnm_W1'], r2(p['fnm_b1']),
        p['fnm_W2'], r2(p['fnm_b2']), p['rd_fg_W'], r2(p['rd_fg_b']))

    fpad = 2048 - f
    wf0 = jnp.pad(wf, ((0, fpad), (0, 0)))
    ffl = jnp.pad(new_ff[:, :32], ((0, fpad), (0, 0)),
                  constant_values=NEG_INF)
    ffr = jnp.pad(new_ff[:, 32:], ((0, fpad), (0, 0)),
                  constant_values=NEG_INF)
    fgg = jnp.pad(fg_graph_ids.astype(jnp.int32), (0, fpad))
    gsum, gmax = _segred(2048, 256, 256)(wf0, fgg, ffl, ffr, fgg)
    gmax = gmax.reshape(32, 256, 32)

    return _readout(nsum, nmax, gsum, gmax,
                    p['reg_W1'], r2(p['reg_b1']),
                    p['reg_W2'], r2(p['reg_b2']))
